# Initial kernel scaffold; baseline (speedup 1.0000x reference)
import jax
import jax.numpy as jnp
from jax.experimental import pallas as pl


def kernel(embeddings, instance_labels):
    raise NotImplementedError("write your pallas kernel here")


# speedup vs baseline: 4.4642x; 4.4642x over previous
import functools

import jax
import jax.numpy as jnp
from jax.experimental import pallas as pl
from jax.experimental.pallas import tpu as pltpu

M = 100000
E = 128
K = 33
KP = 64
B = 2000
NB = M // B
DELTA_PULL = 0.5
DELTA_PUSH = 1.5
ALPHA = 1.0
BETA = 1.0
GAMMA = 0.001


def _body(lab_ref, emb_ref, out_ref, sumsT_ref, counts_ref, meansT_ref,
          msq_ref, pulls_ref):
    g = pl.program_id(0)

    lab = lab_ref[0]
    iota_r = jax.lax.broadcasted_iota(jnp.int32, (1, KP), 1)
    onehot = (lab == iota_r).astype(jnp.float32)

    @pl.when(g == 0)
    def _init():
        sumsT_ref[...] = jnp.zeros_like(sumsT_ref)
        counts_ref[...] = jnp.zeros_like(counts_ref)
        pulls_ref[...] = jnp.zeros_like(pulls_ref)

    @pl.when(g < NB)
    def _phase1():
        emb = emb_ref[...]
        sumsT_ref[...] += jax.lax.dot_general(
            emb, onehot, (((0,), (0,)), ((), ())),
            preferred_element_type=jnp.float32)
        counts_ref[...] += jnp.sum(onehot, axis=0, keepdims=True)

    @pl.when(g == NB - 1)
    def _finish_means():
        safe = jnp.maximum(counts_ref[...], 1.0)
        meansT = sumsT_ref[...] / safe
        meansT_ref[...] = meansT
        msq_ref[...] = jnp.sum(meansT * meansT, axis=0, keepdims=True)

    @pl.when(g >= NB)
    def _phase2():
        emb = emb_ref[...]
        e2 = jnp.sum(emb * emb, axis=1, keepdims=True)
        dots = jax.lax.dot_general(
            emb, meansT_ref[...], (((1,), (0,)), ((), ())),
            preferred_element_type=jnp.float32)
        sel = jnp.sum(dots * onehot, axis=1, keepdims=True)
        m2 = jnp.sum(onehot * msq_ref[...], axis=1, keepdims=True)
        d2 = jnp.maximum(e2 - 2.0 * sel + m2, 0.0)
        dist = jnp.sqrt(d2 + 1e-12)
        w = (lab > 0).astype(jnp.float32)
        pull_b = jnp.square(jnp.maximum(dist - DELTA_PULL, 0.0)) * w
        pulls_ref[...] += jax.lax.dot_general(
            pull_b, onehot, (((0,), (0,)), ((), ())),
            preferred_element_type=jnp.float32)

    @pl.when(g == 2 * NB - 1)
    def _final():
        counts = counts_ref[...]
        safe = jnp.maximum(counts, 1.0)
        col_id = jax.lax.broadcasted_iota(jnp.int32, (1, KP), 1)
        valid = (counts > 0.0) & (col_id > 0)
        validf = valid.astype(jnp.float32)
        C = jnp.sum(validf)
        Cs = jnp.maximum(C, 1.0)

        pull_loss = jnp.sum(
            jnp.where(valid, pulls_ref[...] / safe, 0.0)) / Cs

        meansT = meansT_ref[...]
        msq = msq_ref[...]
        G = jax.lax.dot_general(
            meansT, meansT, (((0,), (0,)), ((), ())),
            preferred_element_type=jnp.float32)
        ii = jax.lax.broadcasted_iota(jnp.int32, (KP, KP), 0)
        jj = jax.lax.broadcasted_iota(jnp.int32, (KP, KP), 1)
        eye = (ii == jj).astype(jnp.float32)
        msq_col = jnp.sum(eye * msq, axis=1, keepdims=True)
        valid_col = jnp.sum(eye * validf, axis=1, keepdims=True)
        sq = jnp.maximum(msq_col + msq - 2.0 * G, 0.0)
        pm = valid_col * validf * (ii < jj).astype(jnp.float32)
        d = jnp.sqrt(jnp.where(pm > 0.0, sq, 1.0))
        push = jnp.square(jnp.maximum(2.0 * DELTA_PUSH - d, 0.0))
        n_pairs = jnp.sum(pm)
        push_loss = jnp.where(
            n_pairs > 0.0, jnp.sum(push * pm) / jnp.maximum(n_pairs, 1.0), 0.0)

        mnorm = jnp.sqrt(jnp.where(valid, msq, 1.0))
        reg_loss = jnp.sum(jnp.where(valid, mnorm, 0.0)) / Cs

        total = ALPHA * pull_loss + BETA * push_loss + GAMMA * reg_loss
        out_ref[...] = jnp.broadcast_to(total, (1, 1))


@jax.jit
def kernel(embeddings, instance_labels):
    lab3 = instance_labels.astype(jnp.int32).reshape(NB, B, 1)
    out = pl.pallas_call(
        _body,
        grid=(2 * NB,),
        in_specs=[
            pl.BlockSpec((1, B, 1), lambda g: (g % NB, 0, 0)),
            pl.BlockSpec((B, E), lambda g: (g % NB, 0)),
        ],
        out_specs=pl.BlockSpec((1, 1), lambda g: (0, 0)),
        out_shape=jax.ShapeDtypeStruct((1, 1), jnp.float32),
        scratch_shapes=[
            pltpu.VMEM((E, KP), jnp.float32),
            pltpu.VMEM((1, KP), jnp.float32),
            pltpu.VMEM((E, KP), jnp.float32),
            pltpu.VMEM((1, KP), jnp.float32),
            pltpu.VMEM((1, KP), jnp.float32),
        ],
    )(lab3, embeddings)
    return out.reshape(())


# speedup vs baseline: 5.2650x; 1.1794x over previous
import jax
import jax.numpy as jnp
from jax.experimental import pallas as pl
from jax.experimental.pallas import tpu as pltpu

M = 100000
E = 128
K = 33
KP = 64
B = 5000
NB = M // B
DELTA_PULL = 0.5
DELTA_PUSH = 1.5
ALPHA = 1.0
BETA = 1.0
GAMMA = 0.001


def _body(labc_ref, labr_ref, emb_ref, out_ref, sums_ref, counts_ref,
          meansT_ref, msq_ref, pulls_ref):
    g = pl.program_id(0)

    lab_c = labc_ref[0]
    lab_r = labr_ref[0]
    iota_row = jax.lax.broadcasted_iota(jnp.int32, (1, KP), 1)
    iota_col = jax.lax.broadcasted_iota(jnp.int32, (KP, 1), 0)
    onehot_bk = (lab_c == iota_row).astype(jnp.float32)
    onehot_kb = (iota_col == lab_r).astype(jnp.float32)

    @pl.when(g == 0)
    def _init():
        sums_ref[...] = jnp.zeros_like(sums_ref)
        counts_ref[...] = jnp.zeros_like(counts_ref)
        pulls_ref[...] = jnp.zeros_like(pulls_ref)

    @pl.when(g < NB)
    def _phase1():
        emb = emb_ref[...]
        sums_ref[...] += jax.lax.dot_general(
            onehot_kb, emb, (((1,), (0,)), ((), ())),
            preferred_element_type=jnp.float32)
        counts_ref[...] += jax.lax.dot_general(
            onehot_kb, jnp.ones((B, 1), jnp.float32), (((1,), (0,)), ((), ())),
            preferred_element_type=jnp.float32)

    @pl.when(g == NB - 1)
    def _finish_means():
        safe = jnp.maximum(counts_ref[...], 1.0)
        means = sums_ref[...] / safe
        meansT_ref[...] = jnp.swapaxes(means, 0, 1)
        msq_ref[...] = jnp.sum(means * means, axis=1, keepdims=True)

    @pl.when(g >= NB)
    def _phase2():
        emb = emb_ref[...]
        meansT = meansT_ref[...]
        e2 = jnp.sum(emb * emb, axis=1, keepdims=True)
        dots = jax.lax.dot_general(
            emb, meansT, (((1,), (0,)), ((), ())),
            preferred_element_type=jnp.float32)
        msq_row = jnp.sum(meansT * meansT, axis=0, keepdims=True)
        sel = jnp.sum(dots * onehot_bk, axis=1, keepdims=True)
        m2 = jnp.sum(onehot_bk * msq_row, axis=1, keepdims=True)
        d2 = jnp.maximum(e2 - 2.0 * sel + m2, 0.0)
        dist = jnp.sqrt(d2 + 1e-12)
        w = (lab_c > 0).astype(jnp.float32)
        pull_b = jnp.square(jnp.maximum(dist - DELTA_PULL, 0.0)) * w
        pulls_ref[...] += jax.lax.dot_general(
            onehot_kb, pull_b, (((1,), (0,)), ((), ())),
            preferred_element_type=jnp.float32)

    @pl.when(g == 2 * NB - 1)
    def _final():
        counts = counts_ref[...]
        safe = jnp.maximum(counts, 1.0)
        iota_c = jax.lax.broadcasted_iota(jnp.int32, (KP, 1), 0)
        valid = (counts > 0.0) & (iota_c > 0)
        validf = valid.astype(jnp.float32)
        C = jnp.sum(validf)
        Cs = jnp.maximum(C, 1.0)

        pull_loss = jnp.sum(
            jnp.where(valid, pulls_ref[...] / safe, 0.0)) / Cs

        meansT = meansT_ref[...]
        means = jnp.swapaxes(meansT, 0, 1)
        msq_col = msq_ref[...]
        msq_row = jnp.sum(meansT * meansT, axis=0, keepdims=True)
        G = jax.lax.dot_general(
            means, meansT, (((1,), (0,)), ((), ())),
            preferred_element_type=jnp.float32)
        ii = jax.lax.broadcasted_iota(jnp.int32, (KP, KP), 0)
        jj = jax.lax.broadcasted_iota(jnp.int32, (KP, KP), 1)
        eye = (ii == jj).astype(jnp.float32)
        valid_row = jnp.sum(eye * validf, axis=0, keepdims=True)
        sq = jnp.maximum(msq_col + msq_row - 2.0 * G, 0.0)
        pm = validf * valid_row * (ii < jj).astype(jnp.float32)
        d = jnp.sqrt(jnp.where(pm > 0.0, sq, 1.0))
        push = jnp.square(jnp.maximum(2.0 * DELTA_PUSH - d, 0.0))
        n_pairs = jnp.sum(pm)
        push_loss = jnp.where(
            n_pairs > 0.0, jnp.sum(push * pm) / jnp.maximum(n_pairs, 1.0), 0.0)

        mnorm = jnp.sqrt(jnp.where(valid, msq_col, 1.0))
        reg_loss = jnp.sum(jnp.where(valid, mnorm, 0.0)) / Cs

        total = ALPHA * pull_loss + BETA * push_loss + GAMMA * reg_loss
        out_ref[...] = jnp.broadcast_to(total, (1, 1))


@jax.jit
def kernel(embeddings, instance_labels):
    labi = instance_labels.astype(jnp.int32)
    lab_col = labi.reshape(NB, B, 1)
    lab_row = labi.reshape(NB, 1, B)
    out = pl.pallas_call(
        _body,
        grid=(2 * NB,),
        in_specs=[
            pl.BlockSpec((1, B, 1), lambda g: (g % NB, 0, 0)),
            pl.BlockSpec((1, 1, B), lambda g: (g % NB, 0, 0)),
            pl.BlockSpec((B, E), lambda g: (g % NB, 0)),
        ],
        out_specs=pl.BlockSpec((1, 1), lambda g: (0, 0)),
        out_shape=jax.ShapeDtypeStruct((1, 1), jnp.float32),
        scratch_shapes=[
            pltpu.VMEM((KP, E), jnp.float32),
            pltpu.VMEM((KP, 1), jnp.float32),
            pltpu.VMEM((E, KP), jnp.float32),
            pltpu.VMEM((KP, 1), jnp.float32),
            pltpu.VMEM((KP, 1), jnp.float32),
        ],
    )(lab_col, lab_row, embeddings)
    return out.reshape(())


# speedup vs baseline: 6.2864x; 1.1940x over previous
import jax
import jax.numpy as jnp
from jax.experimental import pallas as pl
from jax.experimental.pallas import tpu as pltpu

M = 100000
E = 128
K = 33
KP = 64
B = 5000
NB = M // B
DELTA_PULL = 0.5
DELTA_PUSH = 1.5
ALPHA = 1.0
BETA = 1.0
GAMMA = 0.001


def _body(labc_ref, labr_ref, emb_ref, out_ref, sums_ref, counts_ref,
          meansT_ref, msqh_ref, msq_ref, pulls_ref):
    g = pl.program_id(0)

    @pl.when(g == 0)
    def _init():
        sums_ref[...] = jnp.zeros_like(sums_ref)
        counts_ref[...] = jnp.zeros_like(counts_ref)
        pulls_ref[...] = jnp.zeros_like(pulls_ref)

    @pl.when(g < NB)
    def _phase1():
        lab_r = labr_ref[0]
        iota_col = jax.lax.broadcasted_iota(jnp.int32, (KP, 1), 0)
        onehot_kb = (iota_col == lab_r).astype(jnp.float32)
        emb = emb_ref[...]
        sums_ref[...] += jax.lax.dot_general(
            onehot_kb, emb, (((1,), (0,)), ((), ())),
            preferred_element_type=jnp.float32)
        counts_ref[...] += jnp.sum(onehot_kb, axis=1, keepdims=True)

    @pl.when(g == NB - 1)
    def _finish_means():
        safe = jnp.maximum(counts_ref[...], 1.0)
        means = sums_ref[...] / safe
        meansT = jnp.swapaxes(means, 0, 1)
        meansT_ref[...] = meansT
        msq_ref[...] = jnp.sum(means * means, axis=1, keepdims=True)
        msqh_ref[...] = 0.5 * jnp.sum(meansT * meansT, axis=0,
                                      keepdims=True)

    @pl.when(g >= NB)
    def _phase2():
        lab_c = labc_ref[0]
        iota_row = jax.lax.broadcasted_iota(jnp.int32, (1, KP), 1)
        onehot_bk = (lab_c == iota_row).astype(jnp.float32)
        emb = emb_ref[...]
        e2 = jnp.sum(emb * emb, axis=1, keepdims=True)
        dots = jax.lax.dot_general(
            emb, meansT_ref[...], (((1,), (0,)), ((), ())),
            preferred_element_type=jnp.float32)
        sel = jnp.sum((dots - msqh_ref[...]) * onehot_bk, axis=1,
                      keepdims=True)
        d2 = jnp.maximum(e2 - 2.0 * sel, 0.0)
        dist = jnp.sqrt(d2 + 1e-12)
        pull_b = jnp.square(jnp.maximum(dist - DELTA_PULL, 0.0))
        pulls_ref[...] += jax.lax.dot_general(
            jnp.ones((1, B), jnp.float32), onehot_bk * pull_b,
            (((1,), (0,)), ((), ())),
            preferred_element_type=jnp.float32)

    @pl.when(g == 2 * NB - 1)
    def _final():
        counts = counts_ref[...]
        safe = jnp.maximum(counts, 1.0)
        iota_c = jax.lax.broadcasted_iota(jnp.int32, (KP, 1), 0)
        valid = (counts > 0.0) & (iota_c > 0)
        validf = valid.astype(jnp.float32)
        C = jnp.sum(validf)
        Cs = jnp.maximum(C, 1.0)

        ii = jax.lax.broadcasted_iota(jnp.int32, (KP, KP), 0)
        jj = jax.lax.broadcasted_iota(jnp.int32, (KP, KP), 1)
        eye = (ii == jj).astype(jnp.float32)
        safe_row = jnp.sum(eye * safe, axis=0, keepdims=True)
        valid_rowf = jnp.sum(eye * validf, axis=0, keepdims=True)

        pull_loss = jnp.sum(
            jnp.where(valid_rowf > 0.0, pulls_ref[...] / safe_row, 0.0)) / Cs

        meansT = meansT_ref[...]
        means = jnp.swapaxes(meansT, 0, 1)
        msq_col = msq_ref[...]
        msq_row = 2.0 * msqh_ref[...]
        G = jax.lax.dot_general(
            means, meansT, (((1,), (0,)), ((), ())),
            preferred_element_type=jnp.float32)
        sq = jnp.maximum(msq_col + msq_row - 2.0 * G, 0.0)
        pm = validf * valid_rowf * (ii < jj).astype(jnp.float32)
        d = jnp.sqrt(jnp.where(pm > 0.0, sq, 1.0))
        push = jnp.square(jnp.maximum(2.0 * DELTA_PUSH - d, 0.0))
        n_pairs = jnp.sum(pm)
        push_loss = jnp.where(
            n_pairs > 0.0, jnp.sum(push * pm) / jnp.maximum(n_pairs, 1.0), 0.0)

        mnorm = jnp.sqrt(jnp.where(valid, msq_col, 1.0))
        reg_loss = jnp.sum(jnp.where(valid, mnorm, 0.0)) / Cs

        total = ALPHA * pull_loss + BETA * push_loss + GAMMA * reg_loss
        out_ref[...] = jnp.broadcast_to(total, (1, 1))


@jax.jit
def kernel(embeddings, instance_labels):
    labi = instance_labels.astype(jnp.int32)
    lab_col = labi.reshape(NB, B, 1)
    lab_row = labi.reshape(NB, 1, B)
    out = pl.pallas_call(
        _body,
        grid=(2 * NB,),
        in_specs=[
            pl.BlockSpec((1, B, 1), lambda g: (g % NB, 0, 0)),
            pl.BlockSpec((1, 1, B), lambda g: (g % NB, 0, 0)),
            pl.BlockSpec((B, E), lambda g: (g % NB, 0)),
        ],
        out_specs=pl.BlockSpec((1, 1), lambda g: (0, 0)),
        out_shape=jax.ShapeDtypeStruct((1, 1), jnp.float32),
        scratch_shapes=[
            pltpu.VMEM((KP, E), jnp.float32),
            pltpu.VMEM((KP, 1), jnp.float32),
            pltpu.VMEM((E, KP), jnp.float32),
            pltpu.VMEM((1, KP), jnp.float32),
            pltpu.VMEM((KP, 1), jnp.float32),
            pltpu.VMEM((1, KP), jnp.float32),
        ],
    )(lab_col, lab_row, embeddings)
    return out.reshape(())
